# Initial kernel scaffold; baseline (speedup 1.0000x reference)
import functools

import jax
import jax.numpy as jnp
from jax import lax
from jax.experimental import pallas as pl
from jax.experimental.pallas import tpu as pltpu
from jax.experimental.pallas import tpu_sc as plsc

N_LABELS = 16384
VOCAB = 17
D = 128
HBINS = 32

NC = 2
NS = 16
L = 16

PER_W = N_LABELS // (NC * NS)
NVEC = PER_W // L
NCOL = D // L


@functools.partial(
    pl.kernel,
    out_type=jax.ShapeDtypeStruct((NC, D), jnp.float32),
    mesh=plsc.VectorSubcoreMesh(
        core_axis_name="c", subcore_axis_name="s", num_cores=NC, num_subcores=NS
    ),
    scratch_types=[
        pltpu.VMEM((PER_W,), jnp.int32),
        pltpu.VMEM((HBINS,), jnp.float32),
        pltpu.VMEM((NS, HBINS), jnp.float32),
        pltpu.VMEM((VOCAB, D), jnp.float32),
        pltpu.VMEM((D, D), jnp.float32),
        pltpu.VMEM((NCOL, L), jnp.float32),
        pltpu.VMEM((L,), jnp.float32),
        pltpu.VMEM_SHARED((NS, HBINS), jnp.float32),
        pltpu.VMEM_SHARED((NCOL, L), jnp.float32),
    ],
)
def _sc_contract(idx_hbm, table_hbm, w_hbm, out_hbm,
                 idx_v, hist_v, hists_v, table_v, w_v, pooled_v, stage_v,
                 sh_hists, sh_pooled):
    c = lax.axis_index("c")
    s = lax.axis_index("s")
    base = (c * NS + s) * PER_W

    zeros = jnp.zeros((L,), jnp.float32)

    hist_v[pl.ds(0, L)] = zeros
    hist_v[pl.ds(L, L)] = zeros
    pltpu.sync_copy(idx_hbm.at[pl.ds(base, PER_W)], idx_v)
    ones = jnp.full((L,), 1.0 / N_LABELS, jnp.float32)
    for i in range(NVEC):
        iv = idx_v[pl.ds(i * L, L)]
        plsc.addupdate_scatter(hist_v, [iv], ones)
    pltpu.sync_copy(hist_v, sh_hists.at[s])

    plsc.subcore_barrier()

    col = s * L

    @pl.when(s < NCOL)
    def _stage1():
        pltpu.sync_copy(sh_hists, hists_v)
        tot0 = zeros
        tot1 = zeros
        for w in range(NS):
            tot0 = tot0 + hists_v[w, pl.ds(0, L)]
            tot1 = tot1 + hists_v[w, pl.ds(L, L)]
        hist_v[pl.ds(0, L)] = tot0
        hist_v[pl.ds(L, L)] = tot1
        pltpu.sync_copy(table_hbm, table_v)
        acc = zeros
        for v in range(VOCAB):
            acc = acc + hist_v[v] * table_v[v, pl.ds(col, L)]
        stage_v[...] = acc
        pltpu.sync_copy(stage_v, sh_pooled.at[s])

    plsc.subcore_barrier()

    @pl.when(s < NCOL)
    def _stage2():
        pltpu.sync_copy(sh_pooled, pooled_v)
        pltpu.sync_copy(w_hbm, w_v)
        acc = zeros
        for kc in range(NCOL):
            for kl in range(L):
                acc = acc + pooled_v[kc, kl] * w_v[kc * L + kl, pl.ds(col, L)]
        stage_v[...] = acc
        pltpu.sync_copy(stage_v, out_hbm.at[c, pl.ds(col, L)])


def kernel(indices, table, W):
    parts = _sc_contract(indices.astype(jnp.int32), table, W)
    return parts.sum(axis=0)


# speedup vs baseline: 1.7523x; 1.7523x over previous
import functools

import jax
import jax.numpy as jnp
from jax import lax
from jax.experimental import pallas as pl
from jax.experimental.pallas import tpu as pltpu
from jax.experimental.pallas import tpu_sc as plsc

N_LABELS = 16384
VOCAB = 17
D = 128

NC = 2
NS = 16
L = 16
NW = NC * NS

PER_W = N_LABELS // NW
NVEC = PER_W // L
NCOL = D // L


@functools.partial(
    pl.kernel,
    out_type=jax.ShapeDtypeStruct((NW, D), jnp.float32),
    mesh=plsc.VectorSubcoreMesh(
        core_axis_name="c", subcore_axis_name="s", num_cores=NC, num_subcores=NS
    ),
    compiler_params=pltpu.CompilerParams(needs_layout_passes=False),
    scratch_types=[
        pltpu.VMEM((PER_W,), jnp.int32),
        pltpu.VMEM((VOCAB, D), jnp.float32),
        pltpu.VMEM((D, D), jnp.float32),
        pltpu.VMEM((D,), jnp.float32),
        pltpu.VMEM((D,), jnp.float32),
    ],
)
def _sc_contract(idx_hbm, table_hbm, w_hbm, out_hbm,
                 idx_v, table_v, w_v, pooled_v, outst_v):
    c = lax.axis_index("c")
    s = lax.axis_index("s")
    wid = c * NS + s
    base = wid * PER_W

    zeros = jnp.zeros((L,), jnp.float32)

    pltpu.sync_copy(idx_hbm.at[pl.ds(base, PER_W)], idx_v)
    lanes = lax.iota(jnp.int32, L)
    h0 = jnp.zeros((L,), jnp.int32)
    h1 = jnp.zeros((L,), jnp.int32)
    for i in range(NVEC):
        iv = idx_v[pl.ds(i * L, L)]
        for v in range(L):
            pc = plsc.all_reduce_population_count(iv == v)
            h0 = jnp.where(lanes == v, h0 + pc, h0)
        pc = plsc.all_reduce_population_count(iv == L)
        h1 = jnp.where(lanes == 0, h1 + pc, h1)
    scale = 1.0 / N_LABELS
    tot0 = h0.astype(jnp.float32) * scale
    tot1 = h1.astype(jnp.float32) * scale

    pltpu.sync_copy(table_hbm, table_v)
    for cc in range(NCOL):
        col = cc * L
        acc = zeros
        for v in range(L):
            acc = acc + tot0[v] * table_v[v, pl.ds(col, L)]
        acc = acc + tot1[0] * table_v[L, pl.ds(col, L)]
        pooled_v[pl.ds(col, L)] = acc

    pltpu.sync_copy(w_hbm, w_v)
    for cc in range(NCOL):
        col = cc * L
        acc = zeros
        for kc in range(NCOL):
            pr = pooled_v[pl.ds(kc * L, L)]
            for kl in range(L):
                acc = acc + pr[kl] * w_v[kc * L + kl, pl.ds(col, L)]
        outst_v[pl.ds(col, L)] = acc
    pltpu.sync_copy(outst_v, out_hbm.at[wid])


def kernel(indices, table, W):
    parts = _sc_contract(indices.astype(jnp.int32), table, W)
    return parts.sum(axis=0)


# speedup vs baseline: 1.8393x; 1.0496x over previous
import functools

import jax
import jax.numpy as jnp
from jax import lax
from jax.experimental import pallas as pl
from jax.experimental.pallas import tpu as pltpu
from jax.experimental.pallas import tpu_sc as plsc

N_LABELS = 16384
VOCAB = 17
D = 128

NC = 2
NS = 16
L = 16
NW = NC * NS

PER_W = N_LABELS // NW
NVEC = PER_W // L
NCOL = D // L


@functools.partial(
    pl.kernel,
    out_type=jax.ShapeDtypeStruct((NW, D), jnp.float32),
    mesh=plsc.VectorSubcoreMesh(
        core_axis_name="c", subcore_axis_name="s", num_cores=NC, num_subcores=NS
    ),
    compiler_params=pltpu.CompilerParams(needs_layout_passes=False),
    scratch_types=[
        pltpu.VMEM((PER_W,), jnp.int32),
        pltpu.VMEM((2 * L,), jnp.float32),
        pltpu.VMEM((VOCAB, D), jnp.float32),
        pltpu.VMEM((D, D), jnp.float32),
        pltpu.VMEM((D,), jnp.float32),
        pltpu.VMEM((D,), jnp.float32),
    ],
)
def _sc_contract(idx_hbm, table_hbm, w_hbm, out_hbm,
                 idx_v, hist_v, table_v, w_v, pooled_v, outst_v):
    c = lax.axis_index("c")
    s = lax.axis_index("s")
    wid = c * NS + s
    base = wid * PER_W

    zeros = jnp.zeros((L,), jnp.float32)

    hist_v[pl.ds(0, L)] = zeros
    hist_v[pl.ds(L, L)] = zeros
    pltpu.sync_copy(idx_hbm.at[pl.ds(base, PER_W)], idx_v)
    ones = jnp.full((L,), 1.0 / N_LABELS, jnp.float32)
    for i in range(NVEC):
        iv = idx_v[pl.ds(i * L, L)]
        plsc.addupdate_scatter(hist_v, [iv], ones)
    tot0 = hist_v[pl.ds(0, L)]
    tot1 = hist_v[pl.ds(L, L)]

    pltpu.sync_copy(table_hbm, table_v)
    for cc in range(NCOL):
        col = cc * L
        acc = zeros
        for v in range(L):
            acc = acc + tot0[v] * table_v[v, pl.ds(col, L)]
        acc = acc + tot1[0] * table_v[L, pl.ds(col, L)]
        pooled_v[pl.ds(col, L)] = acc

    pltpu.sync_copy(w_hbm, w_v)
    for cc in range(NCOL):
        col = cc * L
        acc = zeros
        for kc in range(NCOL):
            pr = pooled_v[pl.ds(kc * L, L)]
            for kl in range(L):
                acc = acc + pr[kl] * w_v[kc * L + kl, pl.ds(col, L)]
        outst_v[pl.ds(col, L)] = acc
    pltpu.sync_copy(outst_v, out_hbm.at[wid])


def kernel(indices, table, W):
    parts = _sc_contract(indices.astype(jnp.int32), table, W)
    return parts.sum(axis=0)


# speedup vs baseline: 2.1185x; 1.1518x over previous
import functools

import jax
import jax.numpy as jnp
from jax import lax
from jax.experimental import pallas as pl
from jax.experimental.pallas import tpu as pltpu
from jax.experimental.pallas import tpu_sc as plsc

N_LABELS = 16384
VOCAB = 17
D = 128

NC = 2
NS = 16
L = 16
NW = NC * NS

PER_W = N_LABELS // NW
NVEC = PER_W // L
NCOL = D // L


@functools.partial(
    pl.kernel,
    out_type=jax.ShapeDtypeStruct((NC, NCOL, L), jnp.float32),
    mesh=plsc.VectorSubcoreMesh(
        core_axis_name="c", subcore_axis_name="s", num_cores=NC, num_subcores=NS
    ),
    compiler_params=pltpu.CompilerParams(needs_layout_passes=False),
    scratch_types=[
        pltpu.VMEM((PER_W,), jnp.int32),
        pltpu.VMEM((2 * L,), jnp.float32),
        pltpu.VMEM((VOCAB, D), jnp.float32),
        pltpu.VMEM((D, D), jnp.float32),
        pltpu.VMEM((D,), jnp.float32),
        pltpu.VMEM((NS, D), jnp.float32),
        pltpu.VMEM((L,), jnp.float32),
        pltpu.VMEM_SHARED((NW, D), jnp.float32),
        pltpu.SemaphoreType.DMA,
    ],
)
def _sc_contract(idx_hbm, table_hbm, w_hbm, out_hbm,
                 idx_v, hist_v, table_v, w_v, pooled_v, pools_v, outst_v,
                 sh_pool, wsem):
    c = lax.axis_index("c")
    s = lax.axis_index("s")
    row = c * NS + s
    base = row * PER_W

    zeros = jnp.zeros((L,), jnp.float32)

    wcopy = pltpu.async_copy(w_hbm, w_v, wsem)

    hist_v[pl.ds(0, L)] = zeros
    hist_v[pl.ds(L, L)] = zeros
    pltpu.sync_copy(idx_hbm.at[pl.ds(base, PER_W)], idx_v)
    ones = jnp.full((L,), 1.0 / N_LABELS, jnp.float32)
    for i in range(NVEC):
        iv = idx_v[pl.ds(i * L, L)]
        plsc.addupdate_scatter(hist_v, [iv], ones)
    tot0 = hist_v[pl.ds(0, L)]
    tot1 = hist_v[pl.ds(L, L)]

    pltpu.sync_copy(table_hbm, table_v)
    w0 = [tot0[v] for v in range(L)]
    w16 = tot1[0]
    for cc in range(NCOL):
        col = cc * L
        acc = zeros
        for v in range(L):
            acc = acc + w0[v] * table_v[v, pl.ds(col, L)]
        acc = acc + w16 * table_v[L, pl.ds(col, L)]
        pooled_v[pl.ds(col, L)] = acc
    pltpu.sync_copy(pooled_v, sh_pool.at[row])

    plsc.subcore_barrier()
    wcopy.wait()

    @pl.when(s < NCOL)
    def _stage():
        pltpu.sync_copy(sh_pool.at[pl.ds(c * NS, NS)], pools_v)
        col = s * L
        acc = zeros
        for kc in range(NCOL):
            tp = zeros
            for w in range(NS):
                tp = tp + pools_v[w, pl.ds(kc * L, L)]
            for kl in range(L):
                acc = acc + tp[kl] * w_v[kc * L + kl, pl.ds(col, L)]
        outst_v[...] = acc
        pltpu.sync_copy(outst_v, out_hbm.at[c, s])


def kernel(indices, table, W):
    parts = _sc_contract(indices.astype(jnp.int32), table, W)
    return parts.reshape(NC, D).sum(axis=0)


# speedup vs baseline: 2.3731x; 1.1202x over previous
import functools

import jax
import jax.numpy as jnp
from jax import lax
from jax.experimental import pallas as pl
from jax.experimental.pallas import tpu as pltpu
from jax.experimental.pallas import tpu_sc as plsc

N_LABELS = 16384
VOCAB = 17
D = 128

NC = 1
NS = 16
L = 16
NW = NC * NS

PER_W = N_LABELS // NW
NVEC = PER_W // L
NCOL = D // L


@functools.partial(
    pl.kernel,
    out_type=jax.ShapeDtypeStruct((NC, NCOL, L), jnp.float32),
    mesh=plsc.VectorSubcoreMesh(
        core_axis_name="c", subcore_axis_name="s", num_cores=NC, num_subcores=NS
    ),
    compiler_params=pltpu.CompilerParams(needs_layout_passes=False),
    scratch_types=[
        pltpu.VMEM((PER_W,), jnp.int32),
        pltpu.VMEM((2 * L,), jnp.float32),
        pltpu.VMEM((VOCAB, D), jnp.float32),
        pltpu.VMEM((D, D), jnp.float32),
        pltpu.VMEM((D,), jnp.float32),
        pltpu.VMEM((NS, D), jnp.float32),
        pltpu.VMEM((L,), jnp.float32),
        pltpu.VMEM_SHARED((NW, D), jnp.float32),
        pltpu.SemaphoreType.DMA,
    ],
)
def _sc_contract(idx_hbm, table_hbm, w_hbm, out_hbm,
                 idx_v, hist_v, table_v, w_v, pooled_v, pools_v, outst_v,
                 sh_pool, wsem):
    c = lax.axis_index("c")
    s = lax.axis_index("s")
    row = c * NS + s
    base = row * PER_W

    zeros = jnp.zeros((L,), jnp.float32)

    wcopy = pltpu.async_copy(w_hbm, w_v, wsem)

    hist_v[pl.ds(0, L)] = zeros
    hist_v[pl.ds(L, L)] = zeros
    pltpu.sync_copy(idx_hbm.at[pl.ds(base, PER_W)], idx_v)
    ones = jnp.full((L,), 1.0 / N_LABELS, jnp.float32)
    for i in range(NVEC):
        iv = idx_v[pl.ds(i * L, L)]
        plsc.addupdate_scatter(hist_v, [iv], ones)
    tot0 = hist_v[pl.ds(0, L)]
    tot1 = hist_v[pl.ds(L, L)]

    pltpu.sync_copy(table_hbm, table_v)
    w0 = [tot0[v] for v in range(L)]
    w16 = tot1[0]
    for cc in range(NCOL):
        col = cc * L
        acc = zeros
        for v in range(L):
            acc = acc + w0[v] * table_v[v, pl.ds(col, L)]
        acc = acc + w16 * table_v[L, pl.ds(col, L)]
        pooled_v[pl.ds(col, L)] = acc
    pltpu.sync_copy(pooled_v, sh_pool.at[row])

    plsc.subcore_barrier()
    wcopy.wait()

    @pl.when(s < NCOL)
    def _stage():
        pltpu.sync_copy(sh_pool.at[pl.ds(c * NS, NS)], pools_v)
        col = s * L
        acc = zeros
        for kc in range(NCOL):
            tp = zeros
            for w in range(NS):
                tp = tp + pools_v[w, pl.ds(kc * L, L)]
            for kl in range(L):
                acc = acc + tp[kl] * w_v[kc * L + kl, pl.ds(col, L)]
        outst_v[...] = acc
        pltpu.sync_copy(outst_v, out_hbm.at[c, s])


def kernel(indices, table, W):
    parts = _sc_contract(indices.astype(jnp.int32), table, W)
    return parts.reshape(NC, D).sum(axis=0)
